# baseline (speedup 1.0000x reference)
import dataclasses
import functools

import jax
import jax.numpy as jnp
from jax import lax
from jax.experimental import pallas as pl
from jax.experimental.pallas import tpu as pltpu
from jax.experimental.pallas import tpu_sc as plsc

N = 10000
E = 320000
D = 128
NC = 2
NS = 16
NW = NC * NS
CHD = 128
CD = 80
HP = 8
NB = 8
CHA = 32
ZB = 64
EPAD = NW * CD * CHD
NPAD = 10240
HR = NPAD // D
RPW = NPAD // NS
ZR = 16


def _mesh():
    return plsc.VectorSubcoreMesh(core_axis_name="c", subcore_axis_name="s")


def _no_layout_params():
    cp = pltpu.CompilerParams()
    if "needs_layout_passes" in pltpu.CompilerParams.__dataclass_fields__:
        cp = dataclasses.replace(cp, needs_layout_passes=False)
    return cp


def _sc_degree(dst_rows):
    @functools.partial(
        pl.kernel,
        out_type=jax.ShapeDtypeStruct((NC * HR, D), jnp.float32),
        mesh=_mesh(),
        compiler_params=_no_layout_params(),
        scratch_types=[
            pltpu.VMEM((CD, CHD), jnp.int32),
            pltpu.VMEM((HR, D), jnp.float32),
            pltpu.VMEM((8, D), jnp.float32),
            pltpu.VMEM((HR,), jnp.int32),
            pltpu.VMEM_SHARED((HR, D), jnp.float32),
            pltpu.SemaphoreType.DMA,
        ],
    )
    def deg_kernel(dst_hbm, out_hbm, dstv, hist, zerov, rowidx, sph, sem):
        c = lax.axis_index("c")
        s = lax.axis_index("s")
        w = c * NS + s

        pltpu.async_copy(dst_hbm.at[pl.ds(w * CD, CD)], dstv, sem)

        @pl.loop(0, HR)
        def _(i):
            @pl.loop(0, D, step=16)
            def _(k):
                hist[i, pl.ds(k, 16)] = jnp.zeros((16,), jnp.float32)

        @pl.loop(0, 8)
        def _(i):
            @pl.loop(0, D, step=16)
            def _(k):
                zerov[i, pl.ds(k, 16)] = jnp.zeros((16,), jnp.float32)

        @pl.loop(0, HR, step=16)
        def _(k):
            rowidx[pl.ds(k, 16)] = lax.iota(jnp.int32, 16) + k

        @pl.when(s < HR // 8)
        def _():
            pltpu.sync_copy(zerov, sph.at[pl.ds(s * 8, 8)])

        pltpu.make_async_copy(dst_hbm.at[pl.ds(w * CD, CD)], dstv, sem).wait()
        plsc.subcore_barrier()

        @pl.loop(0, CD)
        def _(j):
            @pl.loop(0, CHD, step=16)
            def _(k):
                v = dstv[j, pl.ds(k, 16)]
                cnt, last = plsc.scan_count(v)
                plsc.addupdate_scatter(
                    hist,
                    [v >> 7, v & 127],
                    cnt.astype(jnp.float32),
                    mask=last,
                )

        pltpu.sync_copy(hist, sph.at[rowidx], add=True)
        plsc.subcore_barrier()

        @pl.when(s < HR // 8)
        def _():
            pltpu.sync_copy(
                sph.at[pl.ds(s * 8, 8)],
                out_hbm.at[pl.ds(c * HR + s * 8, 8)],
            )

    return deg_kernel(dst_rows)


def _sc_aggregate(y, zeros_hbm, src_rows, dst_rows):
    CP = (CD * CHD) // (HP * CHA)

    @functools.partial(
        pl.kernel,
        out_type=jax.ShapeDtypeStruct((NC * NPAD, D), jnp.float32),
        mesh=_mesh(),
        scratch_types=[
            pltpu.VMEM((CP, CHA), jnp.int32),
            pltpu.VMEM((CP, CHA), jnp.int32),
        ]
        + [pltpu.VMEM((CHA, D), jnp.float32) for _ in range(NB)]
        + [
            pltpu.VMEM_SHARED((NPAD, D), jnp.float32),
            pltpu.SemaphoreType.DMA,
        ]
        + [pltpu.SemaphoreType.DMA for _ in range(NB)]
        + [pltpu.SemaphoreType.DMA for _ in range(NB)],
    )
    def agg_kernel(y_hbm, z_hbm, src_hbm, dst_hbm, out_hbm, srcv, dstv, *rest):
        rows = rest[:NB]
        aggs = rest[NB]
        zsem = rest[NB + 1]
        gsems = rest[NB + 2 : NB + 2 + NB]
        ssems = rest[NB + 2 + NB :]
        c = lax.axis_index("c")
        s = lax.axis_index("s")
        w = c * NS + s

        @pl.loop(0, RPW // ZB)
        def _(i):
            pltpu.async_copy(z_hbm, aggs.at[pl.ds(s * RPW + i * ZB, ZB)], zsem)

        @pl.loop(0, RPW // ZB)
        def _(i):
            pltpu.make_async_copy(
                z_hbm, aggs.at[pl.ds(s * RPW + i * ZB, ZB)], zsem
            ).wait()

        plsc.subcore_barrier()

        for p in range(HP):
            base = w * CP * HP + p * CP
            pltpu.sync_copy(src_hbm.at[pl.ds(base, CP)], srcv)
            pltpu.sync_copy(dst_hbm.at[pl.ds(base, CP)], dstv)

            for b in range(NB):
                pltpu.async_copy(y_hbm.at[srcv.at[b]], rows[b], gsems[b])

            @pl.loop(0, CP, step=NB)
            def _(j):
                for b in range(NB):
                    pltpu.make_async_copy(
                        y_hbm.at[srcv.at[j + b]], rows[b], gsems[b]
                    ).wait()
                    pltpu.async_copy(
                        rows[b], aggs.at[dstv.at[j + b]], ssems[b], add=True
                    )
                for b in range(NB):
                    pltpu.make_async_copy(
                        rows[b], aggs.at[dstv.at[j + b]], ssems[b]
                    ).wait()

                    @pl.when(j + NB + b < CP)
                    def _():
                        pltpu.async_copy(
                            y_hbm.at[srcv.at[j + NB + b]], rows[b], gsems[b]
                        )

        plsc.subcore_barrier()
        pltpu.sync_copy(
            aggs.at[pl.ds(s * RPW, RPW)],
            out_hbm.at[pl.ds(c * NPAD + s * RPW, RPW)],
        )

    return agg_kernel(y, zeros_hbm, src_rows, dst_rows)


def _tc_matmul(x, W):
    def body(x_ref, w_ref, o_ref):
        o_ref[...] = jnp.dot(
            x_ref[...], w_ref[...], preferred_element_type=jnp.float32
        )

    return pl.pallas_call(
        body, out_shape=jax.ShapeDtypeStruct((N, D), jnp.float32)
    )(x, W)


def _tc_norm_scale(d0, d1, xw):
    def body(d0_ref, d1_ref, xw_ref, dis_ref, y_ref):
        deg = 1.0 + d0_ref[...] + d1_ref[...]
        dis = lax.rsqrt(deg)
        dis_ref[...] = dis
        y_ref[...] = jnp.broadcast_to(dis, (N, D)) * xw_ref[...]

    return pl.pallas_call(
        body,
        out_shape=[
            jax.ShapeDtypeStruct((N, 1), jnp.float32),
            jax.ShapeDtypeStruct((N, D), jnp.float32),
        ],
    )(d0, d1, xw)


def _tc_layer(agg_parts, y, dis_b, b, W):
    def body(a_ref, y_ref, d_ref, b_ref, w_ref, o_ref):
        d = jnp.broadcast_to(d_ref[...], (N, D))
        a = a_ref[:N, :] + a_ref[NPAD:NPAD + N, :]
        h = jnp.maximum(d * (a + y_ref[...]) + b_ref[...], 0.0)
        o_ref[...] = d * jnp.dot(h, w_ref[...], preferred_element_type=jnp.float32)

    return pl.pallas_call(
        body, out_shape=jax.ShapeDtypeStruct((N, D), jnp.float32)
    )(agg_parts, y, dis_b, b, W)


def _tc_final(agg_parts, y, dis_b, b, Whp, bhp):
    def body(a_ref, y_ref, d_ref, b_ref, w_ref, bo_ref, o_ref):
        a = a_ref[:N, :] + a_ref[NPAD:NPAD + N, :]
        h = jnp.maximum(
            jnp.broadcast_to(d_ref[...], (N, D)) * (a + y_ref[...]) + b_ref[...],
            0.0,
        )
        o_ref[...] = (
            jnp.dot(h, w_ref[...], preferred_element_type=jnp.float32) + bo_ref[...]
        )

    return pl.pallas_call(
        body, out_shape=jax.ShapeDtypeStruct((N, 2), jnp.float32)
    )(agg_parts, y, dis_b, b, Whp, bhp)


def kernel(x, edge_index, W1, b1, W2, b2, Wd, bd, Wp, bp):
    ei = edge_index.astype(jnp.int32)
    src = ei[0]
    dst = ei[1]
    pad = EPAD - E
    fill = jnp.arange(pad, dtype=jnp.int32)
    srcp = jnp.concatenate([src, fill % N]).reshape(NW * CD, CHD)
    dstp = jnp.concatenate([dst, N + fill % (NPAD - N)]).reshape(NW * CD, CHD)
    zblk = jnp.zeros((ZB, D), jnp.float32)

    deg_parts = _sc_degree(dstp)
    d0 = deg_parts[:HR].reshape(NPAD, 1)[:N]
    d1 = deg_parts[HR:].reshape(NPAD, 1)[:N]
    xw1 = _tc_matmul(x, W1)
    dis_b, y1 = _tc_norm_scale(d0, d1, xw1)

    srcpa = srcp.reshape(-1, CHA)
    dstpa = dstp.reshape(-1, CHA)
    a1 = _sc_aggregate(y1, zblk, srcpa, dstpa)
    y2 = _tc_layer(a1, y1, dis_b, b1.reshape(1, D), W2)

    a2 = _sc_aggregate(y2, zblk, srcpa, dstpa)
    whp = jnp.concatenate([Wd, Wp], axis=1)
    bhp = jnp.concatenate([bd, bp]).reshape(1, 2)
    out = _tc_final(a2, y2, dis_b, b2.reshape(1, D), whp, bhp)
    return out[:, :1], out[:, 1:2]


# speedup vs baseline: 1.0563x; 1.0563x over previous
import dataclasses
import functools

import jax
import jax.numpy as jnp
from jax import lax
from jax.experimental import pallas as pl
from jax.experimental.pallas import tpu as pltpu
from jax.experimental.pallas import tpu_sc as plsc

N = 10000
E = 320000
D = 128
NC = 2
NS = 16
NW = NC * NS
CHD = 128
CD = 80
HP = 4
NB = 4
CHA = 64
ZB = 64
EPAD = NW * CD * CHD
NPAD = 10240
HR = NPAD // D
RPW = NPAD // NS
ZR = 16


def _mesh():
    return plsc.VectorSubcoreMesh(core_axis_name="c", subcore_axis_name="s")


def _no_layout_params():
    cp = pltpu.CompilerParams()
    if "needs_layout_passes" in pltpu.CompilerParams.__dataclass_fields__:
        cp = dataclasses.replace(cp, needs_layout_passes=False)
    return cp


def _sc_degree(dst_rows):
    @functools.partial(
        pl.kernel,
        out_type=jax.ShapeDtypeStruct((NC * HR, D), jnp.float32),
        mesh=_mesh(),
        compiler_params=_no_layout_params(),
        scratch_types=[
            pltpu.VMEM((CD, CHD), jnp.int32),
            pltpu.VMEM((HR, D), jnp.float32),
            pltpu.VMEM((8, D), jnp.float32),
            pltpu.VMEM((HR,), jnp.int32),
            pltpu.VMEM_SHARED((HR, D), jnp.float32),
            pltpu.SemaphoreType.DMA,
        ],
    )
    def deg_kernel(dst_hbm, out_hbm, dstv, hist, zerov, rowidx, sph, sem):
        c = lax.axis_index("c")
        s = lax.axis_index("s")
        w = c * NS + s

        pltpu.async_copy(dst_hbm.at[pl.ds(w * CD, CD)], dstv, sem)

        @pl.loop(0, HR)
        def _(i):
            @pl.loop(0, D, step=16)
            def _(k):
                hist[i, pl.ds(k, 16)] = jnp.zeros((16,), jnp.float32)

        @pl.loop(0, 8)
        def _(i):
            @pl.loop(0, D, step=16)
            def _(k):
                zerov[i, pl.ds(k, 16)] = jnp.zeros((16,), jnp.float32)

        @pl.loop(0, HR, step=16)
        def _(k):
            rowidx[pl.ds(k, 16)] = lax.iota(jnp.int32, 16) + k

        @pl.when(s < HR // 8)
        def _():
            pltpu.sync_copy(zerov, sph.at[pl.ds(s * 8, 8)])

        pltpu.make_async_copy(dst_hbm.at[pl.ds(w * CD, CD)], dstv, sem).wait()
        plsc.subcore_barrier()

        @pl.loop(0, CD)
        def _(j):
            @pl.loop(0, CHD, step=16)
            def _(k):
                v = dstv[j, pl.ds(k, 16)]
                cnt, last = plsc.scan_count(v)
                plsc.addupdate_scatter(
                    hist,
                    [v >> 7, v & 127],
                    cnt.astype(jnp.float32),
                    mask=last,
                )

        pltpu.sync_copy(hist, sph.at[rowidx], add=True)
        plsc.subcore_barrier()

        @pl.when(s < HR // 8)
        def _():
            pltpu.sync_copy(
                sph.at[pl.ds(s * 8, 8)],
                out_hbm.at[pl.ds(c * HR + s * 8, 8)],
            )

    return deg_kernel(dst_rows)


def _sc_aggregate(y, zeros_hbm, src_rows, dst_rows):
    CP = (CD * CHD) // (HP * CHA)

    @functools.partial(
        pl.kernel,
        out_type=jax.ShapeDtypeStruct((NC * NPAD, D), jnp.float32),
        mesh=_mesh(),
        scratch_types=[
            pltpu.VMEM((CP, CHA), jnp.int32),
            pltpu.VMEM((CP, CHA), jnp.int32),
        ]
        + [pltpu.VMEM((CHA, D), jnp.float32) for _ in range(NB)]
        + [
            pltpu.VMEM_SHARED((NPAD, D), jnp.float32),
            pltpu.SemaphoreType.DMA,
        ]
        + [pltpu.SemaphoreType.DMA for _ in range(NB)]
        + [pltpu.SemaphoreType.DMA for _ in range(NB)],
    )
    def agg_kernel(y_hbm, z_hbm, src_hbm, dst_hbm, out_hbm, srcv, dstv, *rest):
        rows = rest[:NB]
        aggs = rest[NB]
        zsem = rest[NB + 1]
        gsems = rest[NB + 2 : NB + 2 + NB]
        ssems = rest[NB + 2 + NB :]
        c = lax.axis_index("c")
        s = lax.axis_index("s")
        w = c * NS + s

        base0 = w * CP * HP
        pltpu.async_copy(src_hbm.at[pl.ds(base0, CP)], srcv, ssems[0])
        pltpu.async_copy(dst_hbm.at[pl.ds(base0, CP)], dstv, ssems[1])

        @pl.loop(0, RPW // ZB)
        def _(i):
            pltpu.async_copy(z_hbm, aggs.at[pl.ds(s * RPW + i * ZB, ZB)], zsem)

        pltpu.make_async_copy(src_hbm.at[pl.ds(base0, CP)], srcv, ssems[0]).wait()
        pltpu.make_async_copy(dst_hbm.at[pl.ds(base0, CP)], dstv, ssems[1]).wait()
        for b in range(NB):
            pltpu.async_copy(y_hbm.at[srcv.at[b]], rows[b], gsems[b])

        @pl.loop(0, RPW // ZB)
        def _(i):
            pltpu.make_async_copy(
                z_hbm, aggs.at[pl.ds(s * RPW + i * ZB, ZB)], zsem
            ).wait()

        plsc.subcore_barrier()

        for p in range(HP):
            if p > 0:
                base = w * CP * HP + p * CP
                pltpu.sync_copy(src_hbm.at[pl.ds(base, CP)], srcv)
                pltpu.sync_copy(dst_hbm.at[pl.ds(base, CP)], dstv)

                for b in range(NB):
                    pltpu.async_copy(y_hbm.at[srcv.at[b]], rows[b], gsems[b])

            @pl.loop(0, CP, step=NB)
            def _(j):
                for b in range(NB):
                    pltpu.make_async_copy(
                        y_hbm.at[srcv.at[j + b]], rows[b], gsems[b]
                    ).wait()
                    pltpu.async_copy(
                        rows[b], aggs.at[dstv.at[j + b]], ssems[b], add=True
                    )
                for b in range(NB):
                    pltpu.make_async_copy(
                        rows[b], aggs.at[dstv.at[j + b]], ssems[b]
                    ).wait()

                    @pl.when(j + NB + b < CP)
                    def _():
                        pltpu.async_copy(
                            y_hbm.at[srcv.at[j + NB + b]], rows[b], gsems[b]
                        )

        plsc.subcore_barrier()
        pltpu.sync_copy(
            aggs.at[pl.ds(s * RPW, RPW)],
            out_hbm.at[pl.ds(c * NPAD + s * RPW, RPW)],
        )

    return agg_kernel(y, zeros_hbm, src_rows, dst_rows)


def _tc_matmul(x, W):
    def body(x_ref, w_ref, o_ref):
        o_ref[...] = jnp.dot(
            x_ref[...], w_ref[...], preferred_element_type=jnp.float32
        )

    return pl.pallas_call(
        body, out_shape=jax.ShapeDtypeStruct((N, D), jnp.float32)
    )(x, W)


def _tc_norm_scale(d0, d1, xw):
    def body(d0_ref, d1_ref, xw_ref, dis_ref, y_ref):
        deg = 1.0 + d0_ref[...] + d1_ref[...]
        dis = lax.rsqrt(deg)
        dis_ref[...] = dis
        y_ref[...] = jnp.broadcast_to(dis, (N, D)) * xw_ref[...]

    return pl.pallas_call(
        body,
        out_shape=[
            jax.ShapeDtypeStruct((N, 1), jnp.float32),
            jax.ShapeDtypeStruct((N, D), jnp.float32),
        ],
    )(d0, d1, xw)


def _tc_layer(agg_parts, y, dis_b, b, W):
    def body(a_ref, y_ref, d_ref, b_ref, w_ref, o_ref):
        d = jnp.broadcast_to(d_ref[...], (N, D))
        a = a_ref[:N, :] + a_ref[NPAD:NPAD + N, :]
        h = jnp.maximum(d * (a + y_ref[...]) + b_ref[...], 0.0)
        o_ref[...] = d * jnp.dot(h, w_ref[...], preferred_element_type=jnp.float32)

    return pl.pallas_call(
        body, out_shape=jax.ShapeDtypeStruct((N, D), jnp.float32)
    )(agg_parts, y, dis_b, b, W)


def _tc_final(agg_parts, y, dis_b, b, Whp, bhp):
    def body(a_ref, y_ref, d_ref, b_ref, w_ref, bo_ref, o_ref):
        a = a_ref[:N, :] + a_ref[NPAD:NPAD + N, :]
        h = jnp.maximum(
            jnp.broadcast_to(d_ref[...], (N, D)) * (a + y_ref[...]) + b_ref[...],
            0.0,
        )
        o_ref[...] = (
            jnp.dot(h, w_ref[...], preferred_element_type=jnp.float32) + bo_ref[...]
        )

    return pl.pallas_call(
        body, out_shape=jax.ShapeDtypeStruct((N, 2), jnp.float32)
    )(agg_parts, y, dis_b, b, Whp, bhp)


def kernel(x, edge_index, W1, b1, W2, b2, Wd, bd, Wp, bp):
    ei = edge_index.astype(jnp.int32)
    src = ei[0]
    dst = ei[1]
    pad = EPAD - E
    fill = jnp.arange(pad, dtype=jnp.int32)
    srcp = jnp.concatenate([src, fill % N]).reshape(NW * CD, CHD)
    dstp = jnp.concatenate([dst, N + fill % (NPAD - N)]).reshape(NW * CD, CHD)
    zblk = jnp.zeros((ZB, D), jnp.float32)

    deg_parts = _sc_degree(dstp)
    d0 = deg_parts[:HR].reshape(NPAD, 1)[:N]
    d1 = deg_parts[HR:].reshape(NPAD, 1)[:N]
    xw1 = _tc_matmul(x, W1)
    dis_b, y1 = _tc_norm_scale(d0, d1, xw1)

    srcpa = srcp.reshape(-1, CHA)
    dstpa = dstp.reshape(-1, CHA)
    a1 = _sc_aggregate(y1, zblk, srcpa, dstpa)
    y2 = _tc_layer(a1, y1, dis_b, b1.reshape(1, D), W2)

    a2 = _sc_aggregate(y2, zblk, srcpa, dstpa)
    whp = jnp.concatenate([Wd, Wp], axis=1)
    bhp = jnp.concatenate([bd, bp]).reshape(1, 2)
    out = _tc_final(a2, y2, dis_b, b2.reshape(1, D), whp, bhp)
    return out[:, :1], out[:, 1:2]


# speedup vs baseline: 1.0578x; 1.0015x over previous
import dataclasses
import functools

import jax
import jax.numpy as jnp
from jax import lax
from jax.experimental import pallas as pl
from jax.experimental.pallas import tpu as pltpu
from jax.experimental.pallas import tpu_sc as plsc

N = 10000
E = 320000
D = 128
NC = 2
NS = 16
NW = NC * NS
CHD = 128
CD = 80
HP = 4
NB = 4
CHA = 64
ZB = 64
EPAD = NW * CD * CHD
NPAD = 10240
HR = NPAD // D
RPW = NPAD // NS


def _mesh():
    return plsc.VectorSubcoreMesh(core_axis_name="c", subcore_axis_name="s")


def _no_layout_params():
    cp = pltpu.CompilerParams()
    if "needs_layout_passes" in pltpu.CompilerParams.__dataclass_fields__:
        cp = dataclasses.replace(cp, needs_layout_passes=False)
    return cp


def _sc_degree(dst_rows):
    @functools.partial(
        pl.kernel,
        out_type=jax.ShapeDtypeStruct((NC * HR, D), jnp.float32),
        mesh=_mesh(),
        compiler_params=_no_layout_params(),
        scratch_types=[
            pltpu.VMEM((CD, CHD), jnp.int32),
            pltpu.VMEM((HR, D), jnp.float32),
            pltpu.VMEM((8, D), jnp.float32),
            pltpu.VMEM((HR,), jnp.int32),
            pltpu.VMEM_SHARED((HR, D), jnp.float32),
            pltpu.SemaphoreType.DMA,
        ],
    )
    def deg_kernel(dst_hbm, out_hbm, dstv, hist, zerov, rowidx, sph, sem):
        c = lax.axis_index("c")
        s = lax.axis_index("s")
        w = c * NS + s

        pltpu.async_copy(dst_hbm.at[pl.ds(w * CD, CD)], dstv, sem)

        @pl.loop(0, HR)
        def _(i):
            @pl.loop(0, D, step=16)
            def _(k):
                hist[i, pl.ds(k, 16)] = jnp.zeros((16,), jnp.float32)

        @pl.loop(0, 8)
        def _(i):
            @pl.loop(0, D, step=16)
            def _(k):
                zerov[i, pl.ds(k, 16)] = jnp.zeros((16,), jnp.float32)

        @pl.loop(0, HR, step=16)
        def _(k):
            rowidx[pl.ds(k, 16)] = lax.iota(jnp.int32, 16) + k

        @pl.when(s < HR // 8)
        def _():
            pltpu.sync_copy(zerov, sph.at[pl.ds(s * 8, 8)])

        pltpu.make_async_copy(dst_hbm.at[pl.ds(w * CD, CD)], dstv, sem).wait()
        plsc.subcore_barrier()

        @pl.loop(0, CD)
        def _(j):
            @pl.loop(0, CHD, step=16)
            def _(k):
                v = dstv[j, pl.ds(k, 16)]
                cnt, last = plsc.scan_count(v)
                plsc.addupdate_scatter(
                    hist,
                    [v >> 7, v & 127],
                    cnt.astype(jnp.float32),
                    mask=last,
                )

        pltpu.sync_copy(hist, sph.at[rowidx], add=True)
        plsc.subcore_barrier()

        @pl.when(s < HR // 8)
        def _():
            pltpu.sync_copy(
                sph.at[pl.ds(s * 8, 8)],
                out_hbm.at[pl.ds(c * HR + s * 8, 8)],
            )

    return deg_kernel(dst_rows)


def _sc_aggregate(y, zeros_hbm, src_rows, dst_rows):
    CP = (CD * CHD) // (HP * CHA)

    @functools.partial(
        pl.kernel,
        out_type=jax.ShapeDtypeStruct((NC * NPAD, D), jnp.float32),
        mesh=_mesh(),
        scratch_types=[
            pltpu.VMEM((CP, CHA), jnp.int32),
            pltpu.VMEM((CP, CHA), jnp.int32),
        ]
        + [pltpu.VMEM((CHA, D), jnp.float32) for _ in range(NB)]
        + [
            pltpu.VMEM_SHARED((NPAD, D), jnp.float32),
            pltpu.SemaphoreType.DMA,
        ]
        + [pltpu.SemaphoreType.DMA for _ in range(NB)]
        + [pltpu.SemaphoreType.DMA for _ in range(NB)],
    )
    def agg_kernel(y_hbm, z_hbm, src_hbm, dst_hbm, out_hbm, srcv, dstv, *rest):
        rows = rest[:NB]
        aggs = rest[NB]
        zsem = rest[NB + 1]
        gsems = rest[NB + 2 : NB + 2 + NB]
        ssems = rest[NB + 2 + NB :]
        c = lax.axis_index("c")
        s = lax.axis_index("s")
        w = c * NS + s

        base0 = w * CP * HP
        pltpu.async_copy(src_hbm.at[pl.ds(base0, CP)], srcv, ssems[0])
        pltpu.async_copy(dst_hbm.at[pl.ds(base0, CP)], dstv, ssems[1])

        @pl.loop(0, RPW // ZB)
        def _(i):
            pltpu.async_copy(z_hbm, aggs.at[pl.ds(s * RPW + i * ZB, ZB)], zsem)

        pltpu.make_async_copy(src_hbm.at[pl.ds(base0, CP)], srcv, ssems[0]).wait()
        pltpu.make_async_copy(dst_hbm.at[pl.ds(base0, CP)], dstv, ssems[1]).wait()
        for b in range(NB):
            pltpu.async_copy(y_hbm.at[srcv.at[b]], rows[b], gsems[b])

        @pl.loop(0, RPW // ZB)
        def _(i):
            pltpu.make_async_copy(
                z_hbm, aggs.at[pl.ds(s * RPW + i * ZB, ZB)], zsem
            ).wait()

        plsc.subcore_barrier()

        for p in range(HP):
            if p > 0:
                base = w * CP * HP + p * CP
                pltpu.sync_copy(src_hbm.at[pl.ds(base, CP)], srcv)
                pltpu.sync_copy(dst_hbm.at[pl.ds(base, CP)], dstv)

                for b in range(NB):
                    pltpu.async_copy(y_hbm.at[srcv.at[b]], rows[b], gsems[b])

            @pl.loop(0, CP, step=NB)
            def _(j):
                for b in range(NB):
                    pltpu.make_async_copy(
                        y_hbm.at[srcv.at[j + b]], rows[b], gsems[b]
                    ).wait()
                    pltpu.async_copy(
                        rows[b], aggs.at[dstv.at[j + b]], ssems[b], add=True
                    )
                for b in range(NB):
                    pltpu.make_async_copy(
                        rows[b], aggs.at[dstv.at[j + b]], ssems[b]
                    ).wait()

                    @pl.when(j + NB + b < CP)
                    def _():
                        pltpu.async_copy(
                            y_hbm.at[srcv.at[j + NB + b]], rows[b], gsems[b]
                        )

        plsc.subcore_barrier()
        pltpu.sync_copy(
            aggs.at[pl.ds(s * RPW, RPW)],
            out_hbm.at[pl.ds(c * NPAD + s * RPW, RPW)],
        )

    return agg_kernel(y, zeros_hbm, src_rows, dst_rows)


def _tc_matmul(x, W):
    def body(x_ref, w_ref, o_ref):
        o_ref[...] = jnp.dot(
            x_ref[...], w_ref[...], preferred_element_type=jnp.float32
        )

    return pl.pallas_call(
        body, out_shape=jax.ShapeDtypeStruct((N, D), jnp.float32)
    )(x, W)


def _tc_norm_scale(d0, d1, xw):
    def body(d0_ref, d1_ref, xw_ref, dis_ref, y_ref):
        deg = 1.0 + d0_ref[...] + d1_ref[...]
        dis = lax.rsqrt(deg)
        dis_ref[...] = dis
        y_ref[...] = jnp.broadcast_to(dis, (N, D)) * xw_ref[...]

    return pl.pallas_call(
        body,
        out_shape=[
            jax.ShapeDtypeStruct((N, 1), jnp.float32),
            jax.ShapeDtypeStruct((N, D), jnp.float32),
        ],
    )(d0, d1, xw)


def _tc_layer(agg_parts, y, dis_b, b, W):
    def body(a_ref, y_ref, d_ref, b_ref, w_ref, o_ref):
        d = jnp.broadcast_to(d_ref[...], (N, D))
        a = a_ref[:N, :] + a_ref[NPAD:NPAD + N, :]
        h = jnp.maximum(d * (a + y_ref[...]) + b_ref[...], 0.0)
        o_ref[...] = d * jnp.dot(h, w_ref[...], preferred_element_type=jnp.float32)

    return pl.pallas_call(
        body, out_shape=jax.ShapeDtypeStruct((N, D), jnp.float32)
    )(agg_parts, y, dis_b, b, W)


def _tc_final(agg_parts, y, dis_b, b, Whp, bhp):
    def body(a_ref, y_ref, d_ref, b_ref, w_ref, bo_ref, o_ref):
        a = a_ref[:N, :] + a_ref[NPAD:NPAD + N, :]
        h = jnp.maximum(
            jnp.broadcast_to(d_ref[...], (N, D)) * (a + y_ref[...]) + b_ref[...],
            0.0,
        )
        o_ref[...] = (
            jnp.dot(h, w_ref[...], preferred_element_type=jnp.float32) + bo_ref[...]
        )

    return pl.pallas_call(
        body, out_shape=jax.ShapeDtypeStruct((N, 2), jnp.float32)
    )(agg_parts, y, dis_b, b, Whp, bhp)


def kernel(x, edge_index, W1, b1, W2, b2, Wd, bd, Wp, bp):
    ei = edge_index.astype(jnp.int32)
    src = ei[0]
    dst = ei[1]
    pad = EPAD - E
    fill = jnp.arange(pad, dtype=jnp.int32)
    srcp = jnp.concatenate([src, fill % N]).reshape(NW * CD, CHD)
    dstp = jnp.concatenate([dst, N + fill % (NPAD - N)]).reshape(NW * CD, CHD)
    zblk = jnp.zeros((ZB, D), jnp.float32)

    deg_parts = _sc_degree(dstp)
    d0 = deg_parts[:HR].reshape(NPAD, 1)[:N]
    d1 = deg_parts[HR:].reshape(NPAD, 1)[:N]
    xw1 = _tc_matmul(x, W1)
    dis_b, y1 = _tc_norm_scale(d0, d1, xw1)

    srcpa = srcp.reshape(-1, CHA)
    dstpa = dstp.reshape(-1, CHA)
    a1 = _sc_aggregate(y1, zblk, srcpa, dstpa)
    y2 = _tc_layer(a1, y1, dis_b, b1.reshape(1, D), W2)

    a2 = _sc_aggregate(y2, zblk, srcpa, dstpa)
    whp = jnp.concatenate([Wd, Wp], axis=1)
    bhp = jnp.concatenate([bd, bp]).reshape(1, 2)
    out = _tc_final(a2, y2, dis_b, b2.reshape(1, D), whp, bhp)
    return out[:, :1], out[:, 1:2]
